# baseline (speedup 1.0000x reference)
import jax
import jax.numpy as jnp
from jax.experimental import pallas as pl
from jax.experimental.pallas import tpu as pltpu

BT = 512


def _add_kernel(x_ref, pos_ref, o_ref):
    o_ref[...] = x_ref[...] + pos_ref[...][None, :, :]


def kernel(x, pos_table):
    b, t, d = x.shape
    grid = (t // BT,)
    return pl.pallas_call(
        _add_kernel,
        grid=grid,
        in_specs=[
            pl.BlockSpec((b, BT, d), lambda i: (0, i, 0)),
            pl.BlockSpec((BT, d), lambda i: (i, 0)),
        ],
        out_specs=pl.BlockSpec((b, BT, d), lambda i: (0, i, 0)),
        out_shape=jax.ShapeDtypeStruct((b, t, d), x.dtype),
        compiler_params=pltpu.CompilerParams(
            dimension_semantics=("arbitrary",),
        ),
    )(x, pos_table)


# speedup vs baseline: 1.0068x; 1.0068x over previous
import jax
import jax.numpy as jnp
from jax.experimental import pallas as pl
from jax.experimental.pallas import tpu as pltpu

BT = 2048


def _add_kernel(x_ref, pos_ref, o_ref):
    o_ref[...] = x_ref[...] + pos_ref[...]


def kernel(x, pos_table):
    b, t, d = x.shape
    grid = (t // BT, b)
    return pl.pallas_call(
        _add_kernel,
        grid=grid,
        in_specs=[
            pl.BlockSpec((1, BT, d), lambda i, j: (j, i, 0)),
            pl.BlockSpec((BT, d), lambda i, j: (i, 0)),
        ],
        out_specs=pl.BlockSpec((1, BT, d), lambda i, j: (j, i, 0)),
        out_shape=jax.ShapeDtypeStruct((b, t, d), x.dtype),
        compiler_params=pltpu.CompilerParams(
            dimension_semantics=("parallel", "arbitrary"),
        ),
    )(x, pos_table)
